# baseline (speedup 1.0000x reference)
import functools

import jax
import jax.numpy as jnp
from jax import lax
from jax.experimental import pallas as pl
from jax.experimental.pallas import tpu as pltpu
from jax.experimental.pallas import tpu_sc as plsc

NC = 2
NS = 16
NW = NC * NS


def _gather_body(n_rows, chunk, table_hbm, idx_hbm, out_hbm,
                 idx_v, rows_a, rows_b, sem_a, sem_b):
    wid = lax.axis_index("s") * NC + lax.axis_index("c")
    base = wid * n_rows
    n_chunks = n_rows // chunk

    pltpu.sync_copy(idx_hbm.at[pl.ds(base, n_rows)], idx_v)

    bufs = (rows_a, rows_b)
    sems = (sem_a, sem_b)

    def start_gather(c):
        return pltpu.async_copy(
            table_hbm.at[idx_v.at[pl.ds(c * chunk, chunk)]],
            bufs[c % 2], sems[c % 2])

    pending = start_gather(0)
    for c in range(n_chunks):
        nxt = start_gather(c + 1) if c + 1 < n_chunks else None
        pending.wait()
        pltpu.sync_copy(bufs[c % 2], out_hbm.at[pl.ds(base + c * chunk, chunk)])
        pending = nxt


@jax.jit
def kernel(words, word_emb):
    B, L = words.shape
    V, D = word_emb.shape
    n = B * L
    idx = words.reshape(n).astype(jnp.int32)

    n_rows = n // NW
    chunk = 800
    assert n_rows % chunk == 0

    mesh = plsc.VectorSubcoreMesh(core_axis_name="c", subcore_axis_name="s")
    body = functools.partial(_gather_body, n_rows, chunk)
    out = pl.kernel(
        body,
        out_type=jax.ShapeDtypeStruct((n, D), jnp.float32),
        mesh=mesh,
        compiler_params=pltpu.CompilerParams(use_tc_tiling_on_sc=False),
        scratch_types=[
            pltpu.VMEM((n_rows,), jnp.int32),
            pltpu.VMEM((chunk, D), jnp.float32),
            pltpu.VMEM((chunk, D), jnp.float32),
            pltpu.SemaphoreType.DMA,
            pltpu.SemaphoreType.DMA,
        ],
    )(word_emb, idx)
    return out.reshape(B, L, D)
